# baseline (speedup 1.0000x reference)
import functools

import jax
import jax.numpy as jnp
from jax import lax
from jax.experimental import pallas as pl
from jax.experimental.pallas import tpu as pltpu
from jax.experimental.pallas import tpu_sc as plsc

K = 64
B = 4
N = 4096
F = B * N
H = F // 2
NEG_INV_2SIG2 = -12.5
FB = 4096

NC = 2
NS = 16
NW = NC * NS
FPT = H // NW
CH = 128
NCHUNK = FPT // CH
GROUPS = (CH * 3) // 128
IROWS = NCHUNK * GROUPS


def _g_body(n_ref, a_ref, b_ref, g_ref, x_s, w_s):
    x_s[0:3, :] = n_ref[0]
    X3 = x_s[0:3, :]
    x_s[3:4, :] = jnp.sum(X3 * X3, axis=0, keepdims=True)
    x_s[4:5, :] = jnp.full((1, FB), 1.0, jnp.float32)
    x_s[5:8, :] = jnp.zeros((3, FB), jnp.float32)

    A = jnp.transpose(a_ref[...])
    Bb = jnp.transpose(b_ref[...])
    sa = jnp.sin(A)
    ca = jnp.cos(A)
    wx = sa * jnp.cos(Bb)
    wy = sa * jnp.sin(Bb)
    wz = ca
    wn = wx * wx + wy * wy + wz * wz
    for l in range(4):
        sl = pl.ds(l * K, K)
        w_s[0:1, sl] = 25.0 * wx[l:l + 1, :]
        w_s[1:2, sl] = 25.0 * wy[l:l + 1, :]
        w_s[2:3, sl] = 25.0 * wz[l:l + 1, :]
        w_s[3:4, sl] = jnp.full((1, K), NEG_INV_2SIG2, jnp.float32)
        w_s[4:5, sl] = NEG_INV_2SIG2 * wn[l:l + 1, :]
        w_s[5:8, sl] = jnp.zeros((3, K), jnp.float32)

    e = jnp.exp(lax.dot_general(
        x_s[...], w_s[...], (((0,), (0,)), ((), ())),
        precision=lax.Precision.HIGHEST,
        preferred_element_type=jnp.float32))
    g_ref[...] = (e[:, 0:K] + e[:, K:2 * K]
                  + e[:, 2 * K:3 * K] + e[:, 3 * K:4 * K])


def _compute_g(normals_half, alpha, beta):
    nsteps = H // FB
    per_batch = N // FB
    return pl.pallas_call(
        _g_body,
        grid=(nsteps,),
        in_specs=[
            pl.BlockSpec((1, 3, FB), lambda i: (i // max(per_batch, 1), 0,
                                                i % max(per_batch, 1))),
            pl.BlockSpec((K, 4), lambda i: (0, 0)),
            pl.BlockSpec((K, 4), lambda i: (0, 0)),
        ],
        out_specs=pl.BlockSpec((FB, K), lambda i: (i, 0)),
        out_shape=jax.ShapeDtypeStruct((H, K), jnp.float32),
        scratch_shapes=[
            pltpu.VMEM((8, FB), jnp.float32),
            pltpu.VMEM((8, 4 * K), jnp.float32),
        ],
    )(normals_half, alpha, beta)


def _sc_body(g_hbm, nbr_hbm, out_hbm, idx_v, acc_v, nbr_v, sg0, sg1, sw0, sw1):
    cid = lax.axis_index("c")
    sid = lax.axis_index("s")
    wid = cid * NS + sid
    boff = (wid // (NW // 2)) * N
    tile_base = wid * FPT
    sg = (sg0, sg1)
    sw = (sw0, sw1)

    pltpu.sync_copy(nbr_hbm.at[wid], idx_v)
    for j in range(IROWS):
        for i in range(128 // 16):
            sl = pl.ds(i * 16, 16)
            idx_v[j, sl] = idx_v[j, sl] + boff

    desc = {}
    wb = {}

    def fire(c):
        p = c % 2
        base = tile_base + c * CH
        d = [pltpu.async_copy(g_hbm.at[pl.ds(base, CH)], acc_v.at[p], sg[p])]
        d += [pltpu.async_copy(g_hbm.at[idx_v.at[GROUPS * c + j]],
                               nbr_v.at[p, pl.ds(j * 128, 128)], sg[p])
              for j in range(GROUPS)]
        desc[c] = d

    fire(0)
    for c in range(NCHUNK):
        p = c % 2
        if c + 1 < NCHUNK:
            if c - 1 >= 0:
                wb[c - 1].wait()
            fire(c + 1)
        for d in desc[c]:
            d.wait()

        @plsc.parallel_loop(0, CH, step=1, unroll=8)
        def _acc_loop(f):
            for d in range(K // 16):
                sl = pl.ds(d * 16, 16)
                acc_v[p, f, sl] = (acc_v[p, f, sl] + nbr_v[p, 3 * f, sl]
                                   + nbr_v[p, 3 * f + 1, sl]
                                   + nbr_v[p, 3 * f + 2, sl])

        wb[c] = pltpu.async_copy(acc_v.at[p],
                                 out_hbm.at[pl.ds(tile_base + c * CH, CH)],
                                 sw[p])
    for c in range(max(NCHUNK - 2, 0), NCHUNK):
        wb[c].wait()


@functools.cache
def _sc_gather_sum():
    return pl.kernel(
        _sc_body,
        out_type=jax.ShapeDtypeStruct((H, K), jnp.float32),
        mesh=plsc.VectorSubcoreMesh(core_axis_name="c", subcore_axis_name="s"),
        scratch_types=[
            pltpu.VMEM((IROWS, 128), jnp.int32),
            pltpu.VMEM((2, CH, K), jnp.float32),
            pltpu.VMEM((2, CH * 3, K), jnp.float32),
            pltpu.SemaphoreType.DMA,
            pltpu.SemaphoreType.DMA,
            pltpu.SemaphoreType.DMA,
            pltpu.SemaphoreType.DMA,
        ],
        compiler_params=pltpu.CompilerParams(
            use_tc_tiling_on_sc=False,
            skip_device_barrier=True,
            disable_bounds_checks=True,
            disable_semaphore_checks=True,
        ),
    )


def _bn_body(s0_ref, s1_ref, gm_ref, bt_ref, o_ref):
    s0 = s0_ref[...] * (1.0 / 16.0)
    s1 = s1_ref[...] * (1.0 / 16.0)
    mean = (jnp.sum(s0, axis=0, keepdims=True)
            + jnp.sum(s1, axis=0, keepdims=True)) * (1.0 / F)
    c0 = s0 - mean
    c1 = s1 - mean
    var = (jnp.sum(c0 * c0, axis=0, keepdims=True)
           + jnp.sum(c1 * c1, axis=0, keepdims=True)) * (1.0 / F)
    scale = gm_ref[...] * lax.rsqrt(var + 1e-5)
    shift = bt_ref[...]
    y0 = jnp.maximum(c0 * scale + shift, 0.0)
    y1 = jnp.maximum(c1 * scale + shift, 0.0)
    for b in range(2):
        o_ref[b] = jnp.transpose(y0[b * N:(b + 1) * N, :])
        o_ref[2 + b] = jnp.transpose(y1[b * N:(b + 1) * N, :])


def _bn_relu(s0, s1, gamma, beta):
    return pl.pallas_call(
        _bn_body,
        in_specs=[
            pl.BlockSpec((H, K), lambda: (0, 0)),
            pl.BlockSpec((H, K), lambda: (0, 0)),
            pl.BlockSpec((1, K), lambda: (0, 0)),
            pl.BlockSpec((1, K), lambda: (0, 0)),
        ],
        out_specs=pl.BlockSpec((B, K, N), lambda: (0, 0, 0)),
        out_shape=jax.ShapeDtypeStruct((B, K, N), jnp.float32),
    )(s0, s1, gamma, beta)


@jax.jit
def kernel(normals, neighbor_index, weight_alpha, weight_beta, bn_gamma, bn_beta):
    wa = weight_alpha.reshape(K, 4)
    wb = weight_beta.reshape(K, 4)
    nbr = neighbor_index.reshape(2, NW, IROWS, 128)
    g0 = _compute_g(normals[0:2], wa, wb)
    s0 = _sc_gather_sum()(g0, nbr[0])
    g1 = _compute_g(normals[2:4], wa, wb)
    s1 = _sc_gather_sum()(g1, nbr[1])
    return _bn_relu(s0, s1, bn_gamma.reshape(1, K), bn_beta.reshape(1, K))


# speedup vs baseline: 1.0320x; 1.0320x over previous
import functools

import jax
import jax.numpy as jnp
from jax import lax
from jax.experimental import pallas as pl
from jax.experimental.pallas import tpu as pltpu
from jax.experimental.pallas import tpu_sc as plsc

K = 64
B = 4
N = 4096
F = B * N
H = F
NEG_INV_2SIG2 = -12.5
FB = 4096

NC = 2
NS = 16
NW = NC * NS
FPT = H // NW
CH = 128
NCHUNK = FPT // CH
GROUPS = (CH * 3) // 128
IROWS = NCHUNK * GROUPS


def _g_body(n_ref, a_ref, b_ref, g_ref, x_s, w_s):
    x_s[0:3, :] = n_ref[0]
    X3 = x_s[0:3, :]
    x_s[3:4, :] = jnp.sum(X3 * X3, axis=0, keepdims=True)
    x_s[4:5, :] = jnp.full((1, FB), 1.0, jnp.float32)
    x_s[5:8, :] = jnp.zeros((3, FB), jnp.float32)

    A = jnp.transpose(a_ref[...])
    Bb = jnp.transpose(b_ref[...])
    sa = jnp.sin(A)
    ca = jnp.cos(A)
    wx = sa * jnp.cos(Bb)
    wy = sa * jnp.sin(Bb)
    wz = ca
    wn = wx * wx + wy * wy + wz * wz
    for l in range(4):
        sl = pl.ds(l * K, K)
        w_s[0:1, sl] = 25.0 * wx[l:l + 1, :]
        w_s[1:2, sl] = 25.0 * wy[l:l + 1, :]
        w_s[2:3, sl] = 25.0 * wz[l:l + 1, :]
        w_s[3:4, sl] = jnp.full((1, K), NEG_INV_2SIG2, jnp.float32)
        w_s[4:5, sl] = NEG_INV_2SIG2 * wn[l:l + 1, :]
        w_s[5:8, sl] = jnp.zeros((3, K), jnp.float32)

    e = jnp.exp(lax.dot_general(
        x_s[...], w_s[...], (((0,), (0,)), ((), ())),
        precision=lax.Precision.HIGHEST,
        preferred_element_type=jnp.float32))
    g_ref[...] = (e[:, 0:K] + e[:, K:2 * K]
                  + e[:, 2 * K:3 * K] + e[:, 3 * K:4 * K])


def _compute_g(normals_half, alpha, beta):
    nsteps = H // FB
    per_batch = N // FB
    return pl.pallas_call(
        _g_body,
        grid=(nsteps,),
        in_specs=[
            pl.BlockSpec((1, 3, FB), lambda i: (i // max(per_batch, 1), 0,
                                                i % max(per_batch, 1))),
            pl.BlockSpec((K, 4), lambda i: (0, 0)),
            pl.BlockSpec((K, 4), lambda i: (0, 0)),
        ],
        out_specs=pl.BlockSpec((FB, K), lambda i: (i, 0)),
        out_shape=jax.ShapeDtypeStruct((H, K), jnp.float32),
        scratch_shapes=[
            pltpu.VMEM((8, FB), jnp.float32),
            pltpu.VMEM((8, 4 * K), jnp.float32),
        ],
    )(normals_half, alpha, beta)


def _sc_body(g_hbm, nbr_hbm, out_hbm, idx_v, acc_v, nbr_v, sg0, sg1, sw0, sw1):
    cid = lax.axis_index("c")
    sid = lax.axis_index("s")
    wid = cid * NS + sid
    boff = (wid // (NW // B)) * N
    tile_base = wid * FPT
    sg = (sg0, sg1)
    sw = (sw0, sw1)

    pltpu.sync_copy(nbr_hbm.at[wid], idx_v)
    for j in range(IROWS):
        for i in range(128 // 16):
            sl = pl.ds(i * 16, 16)
            idx_v[j, sl] = idx_v[j, sl] + boff

    desc = {}
    wb = {}

    def fire(c):
        p = c % 2
        base = tile_base + c * CH
        d = [pltpu.async_copy(g_hbm.at[pl.ds(base, CH)], acc_v.at[p], sg[p])]
        d += [pltpu.async_copy(g_hbm.at[idx_v.at[GROUPS * c + j]],
                               nbr_v.at[p, pl.ds(j * 128, 128)], sg[p])
              for j in range(GROUPS)]
        desc[c] = d

    fire(0)
    for c in range(NCHUNK):
        p = c % 2
        if c + 1 < NCHUNK:
            if c - 1 >= 0:
                wb[c - 1].wait()
            fire(c + 1)
        for d in desc[c]:
            d.wait()

        @plsc.parallel_loop(0, CH, step=1, unroll=8)
        def _acc_loop(f):
            for d in range(K // 16):
                sl = pl.ds(d * 16, 16)
                acc_v[p, f, sl] = (acc_v[p, f, sl] + nbr_v[p, 3 * f, sl]
                                   + nbr_v[p, 3 * f + 1, sl]
                                   + nbr_v[p, 3 * f + 2, sl])

        wb[c] = pltpu.async_copy(acc_v.at[p],
                                 out_hbm.at[pl.ds(tile_base + c * CH, CH)],
                                 sw[p])
    for c in range(max(NCHUNK - 2, 0), NCHUNK):
        wb[c].wait()


@functools.cache
def _sc_gather_sum():
    return pl.kernel(
        _sc_body,
        out_type=jax.ShapeDtypeStruct((H, K), jnp.float32),
        mesh=plsc.VectorSubcoreMesh(core_axis_name="c", subcore_axis_name="s"),
        scratch_types=[
            pltpu.VMEM((IROWS, 128), jnp.int32),
            pltpu.VMEM((2, CH, K), jnp.float32),
            pltpu.VMEM((2, CH * 3, K), jnp.float32),
            pltpu.SemaphoreType.DMA,
            pltpu.SemaphoreType.DMA,
            pltpu.SemaphoreType.DMA,
            pltpu.SemaphoreType.DMA,
        ],
        compiler_params=pltpu.CompilerParams(
            use_tc_tiling_on_sc=False,
            skip_device_barrier=True,
            disable_bounds_checks=True,
            disable_semaphore_checks=True,
        ),
    )


def _bn_body(s_ref, gm_ref, bt_ref, o_ref):
    s = s_ref[...] * (1.0 / 16.0)
    mean = jnp.mean(s, axis=0, keepdims=True)
    xc = s - mean
    var = jnp.mean(xc * xc, axis=0, keepdims=True)
    scale = gm_ref[...] * lax.rsqrt(var + 1e-5)
    y = jnp.maximum(xc * scale + bt_ref[...], 0.0)
    for b in range(B):
        o_ref[b] = jnp.transpose(y[b * N:(b + 1) * N, :])


def _bn_relu(s, gamma, beta):
    return pl.pallas_call(
        _bn_body,
        in_specs=[
            pl.BlockSpec((F, K), lambda: (0, 0)),
            pl.BlockSpec((1, K), lambda: (0, 0)),
            pl.BlockSpec((1, K), lambda: (0, 0)),
        ],
        out_specs=pl.BlockSpec((B, K, N), lambda: (0, 0, 0)),
        out_shape=jax.ShapeDtypeStruct((B, K, N), jnp.float32),
    )(s, gamma, beta)


@jax.jit
def kernel(normals, neighbor_index, weight_alpha, weight_beta, bn_gamma, bn_beta):
    wa = weight_alpha.reshape(K, 4)
    wb = weight_beta.reshape(K, 4)
    nbr = neighbor_index.reshape(NW, IROWS, 128)
    g = _compute_g(normals, wa, wb)
    s = _sc_gather_sum()(g, nbr)
    return _bn_relu(s, bn_gamma.reshape(1, K), bn_beta.reshape(1, K))


# speedup vs baseline: 1.0338x; 1.0018x over previous
import functools

import jax
import jax.numpy as jnp
from jax import lax
from jax.experimental import pallas as pl
from jax.experimental.pallas import tpu as pltpu
from jax.experimental.pallas import tpu_sc as plsc

K = 64
B = 4
N = 4096
F = B * N
H = F
NEG_INV_2SIG2 = -12.5
FB = 8192

NC = 2
NS = 16
NW = NC * NS
FPT = H // NW
CH = 128
NCHUNK = FPT // CH
GROUPS = (CH * 3) // 128
IROWS = NCHUNK * GROUPS


def _g_body(n_ref, a_ref, b_ref, g_ref, x_s, w_s):
    for bb in range(FB // N):
        x_s[0:3, bb * N:(bb + 1) * N] = n_ref[bb]
    X3 = x_s[0:3, :]
    x_s[3:4, :] = jnp.sum(X3 * X3, axis=0, keepdims=True)
    x_s[4:5, :] = jnp.full((1, FB), 1.0, jnp.float32)
    x_s[5:8, :] = jnp.zeros((3, FB), jnp.float32)

    A = jnp.transpose(a_ref[...])
    Bb = jnp.transpose(b_ref[...])
    sa = jnp.sin(A)
    ca = jnp.cos(A)
    wx = sa * jnp.cos(Bb)
    wy = sa * jnp.sin(Bb)
    wz = ca
    wn = wx * wx + wy * wy + wz * wz
    for l in range(4):
        sl = pl.ds(l * K, K)
        w_s[0:1, sl] = 25.0 * wx[l:l + 1, :]
        w_s[1:2, sl] = 25.0 * wy[l:l + 1, :]
        w_s[2:3, sl] = 25.0 * wz[l:l + 1, :]
        w_s[3:4, sl] = jnp.full((1, K), NEG_INV_2SIG2, jnp.float32)
        w_s[4:5, sl] = NEG_INV_2SIG2 * wn[l:l + 1, :]
        w_s[5:8, sl] = jnp.zeros((3, K), jnp.float32)

    e = jnp.exp(lax.dot_general(
        x_s[...], w_s[...], (((0,), (0,)), ((), ())),
        precision=lax.Precision.HIGHEST,
        preferred_element_type=jnp.float32))
    g_ref[...] = (e[:, 0:K] + e[:, K:2 * K]
                  + e[:, 2 * K:3 * K] + e[:, 3 * K:4 * K])


def _compute_g(normals_half, alpha, beta):
    nsteps = H // FB
    return pl.pallas_call(
        _g_body,
        grid=(nsteps,),
        in_specs=[
            pl.BlockSpec((FB // N, 3, N), lambda i: (i, 0, 0)),
            pl.BlockSpec((K, 4), lambda i: (0, 0)),
            pl.BlockSpec((K, 4), lambda i: (0, 0)),
        ],
        out_specs=pl.BlockSpec((FB, K), lambda i: (i, 0)),
        out_shape=jax.ShapeDtypeStruct((H, K), jnp.float32),
        scratch_shapes=[
            pltpu.VMEM((8, FB), jnp.float32),
            pltpu.VMEM((8, 4 * K), jnp.float32),
        ],
    )(normals_half, alpha, beta)


def _sc_body(g_hbm, nbr_hbm, out_hbm, idx_v, acc_v, nbr_v, sg0, sg1, sw0, sw1):
    cid = lax.axis_index("c")
    sid = lax.axis_index("s")
    wid = cid * NS + sid
    boff = (wid // (NW // B)) * N
    tile_base = wid * FPT
    sg = (sg0, sg1)
    sw = (sw0, sw1)

    pltpu.sync_copy(nbr_hbm.at[wid], idx_v)
    for j in range(IROWS):
        for i in range(128 // 16):
            sl = pl.ds(i * 16, 16)
            idx_v[j, sl] = idx_v[j, sl] + boff

    desc = {}
    wb = {}

    def fire(c):
        p = c % 2
        base = tile_base + c * CH
        d = [pltpu.async_copy(g_hbm.at[pl.ds(base, CH)], acc_v.at[p], sg[p])]
        d += [pltpu.async_copy(g_hbm.at[idx_v.at[GROUPS * c + j]],
                               nbr_v.at[p, pl.ds(j * 128, 128)], sg[p])
              for j in range(GROUPS)]
        desc[c] = d

    fire(0)
    for c in range(NCHUNK):
        p = c % 2
        if c + 1 < NCHUNK:
            if c - 1 >= 0:
                wb[c - 1].wait()
            fire(c + 1)
        for d in desc[c]:
            d.wait()

        @plsc.parallel_loop(0, CH, step=1, unroll=8)
        def _acc_loop(f):
            for d in range(K // 16):
                sl = pl.ds(d * 16, 16)
                acc_v[p, f, sl] = (acc_v[p, f, sl] + nbr_v[p, 3 * f, sl]
                                   + nbr_v[p, 3 * f + 1, sl]
                                   + nbr_v[p, 3 * f + 2, sl])

        wb[c] = pltpu.async_copy(acc_v.at[p],
                                 out_hbm.at[pl.ds(tile_base + c * CH, CH)],
                                 sw[p])
    for c in range(max(NCHUNK - 2, 0), NCHUNK):
        wb[c].wait()


@functools.cache
def _sc_gather_sum():
    return pl.kernel(
        _sc_body,
        out_type=jax.ShapeDtypeStruct((H, K), jnp.float32),
        mesh=plsc.VectorSubcoreMesh(core_axis_name="c", subcore_axis_name="s"),
        scratch_types=[
            pltpu.VMEM((IROWS, 128), jnp.int32),
            pltpu.VMEM((2, CH, K), jnp.float32),
            pltpu.VMEM((2, CH * 3, K), jnp.float32),
            pltpu.SemaphoreType.DMA,
            pltpu.SemaphoreType.DMA,
            pltpu.SemaphoreType.DMA,
            pltpu.SemaphoreType.DMA,
        ],
        compiler_params=pltpu.CompilerParams(
            use_tc_tiling_on_sc=False,
            skip_device_barrier=True,
            disable_bounds_checks=True,
            disable_semaphore_checks=True,
        ),
    )


def _bn_body(s_ref, gm_ref, bt_ref, o_ref):
    s = s_ref[...] * (1.0 / 16.0)
    mean = jnp.mean(s, axis=0, keepdims=True)
    xc = s - mean
    var = jnp.mean(xc * xc, axis=0, keepdims=True)
    scale = gm_ref[...] * lax.rsqrt(var + 1e-5)
    y = jnp.maximum(xc * scale + bt_ref[...], 0.0)
    for b in range(B):
        o_ref[b] = jnp.transpose(y[b * N:(b + 1) * N, :])


def _bn_relu(s, gamma, beta):
    return pl.pallas_call(
        _bn_body,
        in_specs=[
            pl.BlockSpec((F, K), lambda: (0, 0)),
            pl.BlockSpec((1, K), lambda: (0, 0)),
            pl.BlockSpec((1, K), lambda: (0, 0)),
        ],
        out_specs=pl.BlockSpec((B, K, N), lambda: (0, 0, 0)),
        out_shape=jax.ShapeDtypeStruct((B, K, N), jnp.float32),
    )(s, gamma, beta)


@jax.jit
def kernel(normals, neighbor_index, weight_alpha, weight_beta, bn_gamma, bn_beta):
    wa = weight_alpha.reshape(K, 4)
    wb = weight_beta.reshape(K, 4)
    nbr = neighbor_index.reshape(NW, IROWS, 128)
    g = _compute_g(normals, wa, wb)
    s = _sc_gather_sum()(g, nbr)
    return _bn_relu(s, bn_gamma.reshape(1, K), bn_beta.reshape(1, K))
